# baseline (speedup 1.0000x reference)
import functools

import jax
import jax.numpy as jnp
from jax import lax
from jax.experimental import pallas as pl
from jax.experimental.pallas import tpu as pltpu
from jax.experimental.pallas import tpu_sc as plsc

NODES = 1000000
BATCH = 16384
EMBED = 16
PACK = 8
ROWW = PACK * EMBED

_INFO = plsc.get_sparse_core_info()
NC = _INFO.num_cores
NS = _INFO.num_subcores
L = _INFO.num_lanes
NW = NC * NS
BPW = BATCH // NW
CHUNK = 128
NCHUNK = BPW // CHUNK
GROUPS = CHUNK // L

WIN = 512
NFULL = NODES // 128
NSUP = NODES // WIN
TAILBASE = NSUP * WIN
TPW = (NSUP + NW - 1) // NW
WINW = 16 * WIN
PROWS = NODES // PACK
JMAX = TAILBASE // PACK - 1


def _relayout_body(wt_hbm, wl_out, xb0, xb1, yb0, yb1, si0, si1, so0, so1):
    wid = lax.axis_index("s") * NC + lax.axis_index("c")
    dlane = lax.iota(jnp.int32, L)

    def win(c):
        return wt_hbm.at[:, pl.ds(pl.multiple_of(c * WIN, 128), WIN)]

    def outw(c):
        return wl_out.at[pl.ds(pl.multiple_of(c * WINW, WINW), WINW)]

    def start_in(c, xb, s):
        @pl.when(c < NSUP)
        def _():
            pltpu.async_copy(win(c), xb, s)

    def wait_in(c, xb, s):
        @pl.when(c < NSUP)
        def _():
            pltpu.make_async_copy(win(c), xb, s).wait()

    def permute_and_flush(c, xb, yb, s):
        @pl.when(c < NSUP)
        def _():
            @plsc.parallel_loop(0, WIN, unroll=16)
            def _(i):
                col = plsc.load_gather(xb, [dlane, jnp.full((L,), i, jnp.int32)])
                yb[pl.ds(i * EMBED, EMBED)] = col

            pltpu.async_copy(yb, outw(c), s)

    def wait_out(cond, c, yb, s):
        @pl.when(cond)
        def _():
            pltpu.make_async_copy(yb, outw(jnp.minimum(c, NSUP - 1)), s).wait()

    start_in(wid, xb0, si0)

    def rnd(r, _):
        t0 = 2 * r
        c0 = wid + NW * t0
        c1 = c0 + NW
        c2 = c1 + NW
        start_in(c1, xb1, si1)
        wait_in(c0, xb0, si0)
        wait_out(r > 0, c0, yb0, so0)
        permute_and_flush(c0, xb0, yb0, so0)
        start_in(c2, xb0, si0)
        wait_in(c1, xb1, si1)
        wait_out(r > 0, c1, yb1, so1)
        permute_and_flush(c1, xb1, yb1, so1)
        return 0

    lax.fori_loop(0, (TPW + 1) // 2, rnd, 0)
    ceven = wid + NW * (TPW - 2)
    wait_out(ceven < NSUP, ceven, yb0, so0)
    codd = wid + NW * (TPW - 1)
    wait_out(codd < NSUP, codd, yb1, so1)


def _gather_body(pos_u_hbm, pos_v_hbm, neg_v_hbm, wl_hbm, tail_hbm,
                 pos_out, neg_out,
                 iu, iv, inn, ju, jv, jn, ubuf, vbuf, nbuf, tbuf,
                 acc_p, acc_n, sem):
    wid = lax.axis_index("s") * NC + lax.axis_index("c")
    base = wid * BPW
    lane = lax.iota(jnp.int32, L)
    pltpu.sync_copy(tail_hbm, tbuf)

    def chunk_body(k, _):
        cbase = base + k * CHUNK
        pltpu.sync_copy(pos_u_hbm.at[pl.ds(cbase, CHUNK)], iu)
        pltpu.sync_copy(pos_v_hbm.at[pl.ds(cbase, CHUNK)], iv)
        pltpu.sync_copy(neg_v_hbm.at[pl.ds(cbase, CHUNK)], inn)
        for g in range(GROUPS):
            sl = pl.ds(g * L, L)
            ju[sl] = jnp.minimum(lax.shift_right_logical(iu[sl], 3), JMAX)
            jv[sl] = jnp.minimum(lax.shift_right_logical(iv[sl], 3), JMAX)
            jn[sl] = jnp.minimum(lax.shift_right_logical(inn[sl], 3), JMAX)
        cu = pltpu.async_copy(wl_hbm.at[ju], ubuf, sem)
        cv = pltpu.async_copy(wl_hbm.at[jv], vbuf, sem)
        cn = pltpu.async_copy(wl_hbm.at[jn], nbuf, sem)
        cu.wait()
        cv.wait()
        cn.wait()

        def pick(buf, isl, ridx):
            b = (isl & 7) * EMBED
            tr = jnp.clip(lax.shift_right_logical(isl, 3) - (JMAX + 1),
                          0, PACK - 1)
            m = isl >= TAILBASE

            def at(d):
                return jnp.where(m,
                                 plsc.load_gather(tbuf, [tr, b + d]),
                                 plsc.load_gather(buf, [ridx, b + d]))
            return at

        for g in range(GROUPS):
            sl = pl.ds(g * L, L)
            ridx = g * L + lane
            gu = pick(ubuf, iu[sl], ridx)
            gv = pick(vbuf, iv[sl], ridx)
            gn = pick(nbuf, inn[sl], ridx)
            ap = jnp.zeros((L,), jnp.float32)
            an = jnp.zeros((L,), jnp.float32)
            for d in range(EMBED):
                ud = gu(d)
                ap = ap + ud * gv(d)
                an = an + ud * gn(d)
            acc_p[pl.ds(k * CHUNK + g * L, L)] = ap
            acc_n[pl.ds(k * CHUNK + g * L, L)] = an
        return 0

    lax.fori_loop(0, NCHUNK, chunk_body, 0)
    pltpu.sync_copy(acc_p, pos_out.at[pl.ds(base, BPW)])
    pltpu.sync_copy(acc_n, neg_out.at[pl.ds(base, BPW)])


@jax.jit
def kernel(pos_u, pos_v, neg_v, W):
    pos_u = pos_u.astype(jnp.int32)
    pos_v = pos_v.astype(jnp.int32)
    neg_v = neg_v.astype(jnp.int32)
    wt = W.T
    mesh = plsc.VectorSubcoreMesh(core_axis_name="c", subcore_axis_name="s")

    relayout = functools.partial(
        pl.kernel,
        mesh=mesh,
        compiler_params=pltpu.CompilerParams(
            needs_layout_passes=False, use_tc_tiling_on_sc=True),
        out_type=jax.ShapeDtypeStruct((PROWS * ROWW,), jnp.float32),
        scratch_types=[
            pltpu.VMEM((EMBED, WIN), jnp.float32),
            pltpu.VMEM((EMBED, WIN), jnp.float32),
            pltpu.VMEM((WINW,), jnp.float32),
            pltpu.VMEM((WINW,), jnp.float32),
            pltpu.SemaphoreType.DMA,
            pltpu.SemaphoreType.DMA,
            pltpu.SemaphoreType.DMA,
            pltpu.SemaphoreType.DMA,
        ],
    )(_relayout_body)
    wl = jnp.reshape(relayout(wt), (PROWS, ROWW))
    tailp = jnp.reshape(W[TAILBASE:, :], (PACK, ROWW))

    gather = functools.partial(
        pl.kernel,
        mesh=mesh,
        compiler_params=pltpu.CompilerParams(
            needs_layout_passes=False, use_tc_tiling_on_sc=False),
        out_type=(jax.ShapeDtypeStruct((BATCH,), jnp.float32),
                  jax.ShapeDtypeStruct((BATCH,), jnp.float32)),
        scratch_types=[
            pltpu.VMEM((CHUNK,), jnp.int32),
            pltpu.VMEM((CHUNK,), jnp.int32),
            pltpu.VMEM((CHUNK,), jnp.int32),
            pltpu.VMEM((CHUNK,), jnp.int32),
            pltpu.VMEM((CHUNK,), jnp.int32),
            pltpu.VMEM((CHUNK,), jnp.int32),
            pltpu.VMEM((CHUNK, ROWW), jnp.float32),
            pltpu.VMEM((CHUNK, ROWW), jnp.float32),
            pltpu.VMEM((CHUNK, ROWW), jnp.float32),
            pltpu.VMEM((PACK, ROWW), jnp.float32),
            pltpu.VMEM((BPW,), jnp.float32),
            pltpu.VMEM((BPW,), jnp.float32),
            pltpu.SemaphoreType.DMA,
        ],
    )(_gather_body)
    return gather(pos_u, pos_v, neg_v, wl, tailp)


# speedup vs baseline: 1.0184x; 1.0184x over previous
import functools

import jax
import jax.numpy as jnp
from jax import lax
from jax.experimental import pallas as pl
from jax.experimental.pallas import tpu as pltpu
from jax.experimental.pallas import tpu_sc as plsc

NODES = 1000000
BATCH = 16384
EMBED = 16
PACK = 8
ROWW = PACK * EMBED

_INFO = plsc.get_sparse_core_info()
NC = _INFO.num_cores
NS = _INFO.num_subcores
L = _INFO.num_lanes
NW = NC * NS
BPW = BATCH // NW
CHUNK = 128
NCHUNK = BPW // CHUNK
GROUPS = CHUNK // L

WIN = 512
NFULL = NODES // 128
NSUP = NODES // WIN
TAILBASE = NSUP * WIN
TPW = (NSUP + NW - 1) // NW
WINW = 16 * WIN
PROWS = NODES // PACK
JMAX = TAILBASE // PACK - 1


def _relayout_body(wt_hbm, wl_out, xb0, xb1, yb0, yb1, si0, si1, so0, so1):
    wid = lax.axis_index("s") * NC + lax.axis_index("c")
    dlane = lax.iota(jnp.int32, L)

    def win(c):
        return wt_hbm.at[:, pl.ds(pl.multiple_of(c * WIN, 128), WIN)]

    def outw(c):
        return wl_out.at[pl.ds(pl.multiple_of(c * WINW, WINW), WINW)]

    def start_in(c, xb, s):
        @pl.when(c < NSUP)
        def _():
            pltpu.async_copy(win(c), xb.at[:, pl.ds(0, WIN)], s)

    def wait_in(c, xb, s):
        @pl.when(c < NSUP)
        def _():
            pltpu.make_async_copy(win(c), xb.at[:, pl.ds(0, WIN)], s).wait()

    def permute_and_flush(c, xb, yb, s):
        @pl.when(c < NSUP)
        def _():
            @plsc.parallel_loop(0, WIN, unroll=16)
            def _(i):
                col = plsc.load_gather(xb, [dlane, jnp.full((L,), i, jnp.int32)])
                yb[pl.ds(i * EMBED, EMBED)] = col

            pltpu.async_copy(yb, outw(c), s)

    def wait_out(cond, c, yb, s):
        @pl.when(cond)
        def _():
            pltpu.make_async_copy(yb, outw(jnp.minimum(c, NSUP - 1)), s).wait()

    start_in(wid, xb0, si0)

    def rnd(r, _):
        t0 = 2 * r
        c0 = wid + NW * t0
        c1 = c0 + NW
        c2 = c1 + NW
        start_in(c1, xb1, si1)
        wait_in(c0, xb0, si0)
        wait_out(r > 0, c0, yb0, so0)
        permute_and_flush(c0, xb0, yb0, so0)
        start_in(c2, xb0, si0)
        wait_in(c1, xb1, si1)
        wait_out(r > 0, c1, yb1, so1)
        permute_and_flush(c1, xb1, yb1, so1)
        return 0

    lax.fori_loop(0, (TPW + 1) // 2, rnd, 0)
    ceven = wid + NW * (TPW - 2)
    wait_out(ceven < NSUP, ceven, yb0, so0)
    codd = wid + NW * (TPW - 1)
    wait_out(codd < NSUP, codd, yb1, so1)


def _gather_body(pos_u_hbm, pos_v_hbm, neg_v_hbm, wl_hbm, tail_hbm,
                 pos_out, neg_out,
                 iu, iv, inn, ju, jv, jn, ubuf, vbuf, nbuf, tbuf,
                 acc_p, acc_n, sem):
    wid = lax.axis_index("s") * NC + lax.axis_index("c")
    base = wid * BPW
    lane = lax.iota(jnp.int32, L)
    pltpu.sync_copy(tail_hbm, tbuf)

    def chunk_body(k, _):
        cbase = base + k * CHUNK
        pltpu.sync_copy(pos_u_hbm.at[pl.ds(cbase, CHUNK)], iu)
        pltpu.sync_copy(pos_v_hbm.at[pl.ds(cbase, CHUNK)], iv)
        pltpu.sync_copy(neg_v_hbm.at[pl.ds(cbase, CHUNK)], inn)
        for g in range(GROUPS):
            sl = pl.ds(g * L, L)
            ju[sl] = jnp.minimum(lax.shift_right_logical(iu[sl], 3), JMAX)
            jv[sl] = jnp.minimum(lax.shift_right_logical(iv[sl], 3), JMAX)
            jn[sl] = jnp.minimum(lax.shift_right_logical(inn[sl], 3), JMAX)
        cu = pltpu.async_copy(wl_hbm.at[ju], ubuf, sem)
        cv = pltpu.async_copy(wl_hbm.at[jv], vbuf, sem)
        cn = pltpu.async_copy(wl_hbm.at[jn], nbuf, sem)
        cu.wait()
        cv.wait()
        cn.wait()

        def pick(buf, isl, ridx):
            b = (isl & 7) * EMBED
            tr = jnp.clip(lax.shift_right_logical(isl, 3) - (JMAX + 1),
                          0, PACK - 1)
            m = isl >= TAILBASE

            def at(d):
                return jnp.where(m,
                                 plsc.load_gather(tbuf, [tr, b + d]),
                                 plsc.load_gather(buf, [ridx, b + d]))
            return at

        for g in range(GROUPS):
            sl = pl.ds(g * L, L)
            ridx = g * L + lane
            gu = pick(ubuf, iu[sl], ridx)
            gv = pick(vbuf, iv[sl], ridx)
            gn = pick(nbuf, inn[sl], ridx)
            ap = jnp.zeros((L,), jnp.float32)
            an = jnp.zeros((L,), jnp.float32)
            for step in range(EMBED):
                dv = (lane + step) & (EMBED - 1)
                ud = gu(dv)
                ap = ap + ud * gv(dv)
                an = an + ud * gn(dv)
            acc_p[pl.ds(k * CHUNK + g * L, L)] = ap
            acc_n[pl.ds(k * CHUNK + g * L, L)] = an
        return 0

    lax.fori_loop(0, NCHUNK, chunk_body, 0)
    pltpu.sync_copy(acc_p, pos_out.at[pl.ds(base, BPW)])
    pltpu.sync_copy(acc_n, neg_out.at[pl.ds(base, BPW)])


@jax.jit
def kernel(pos_u, pos_v, neg_v, W):
    pos_u = pos_u.astype(jnp.int32)
    pos_v = pos_v.astype(jnp.int32)
    neg_v = neg_v.astype(jnp.int32)
    wt = W.T
    mesh = plsc.VectorSubcoreMesh(core_axis_name="c", subcore_axis_name="s")

    relayout = functools.partial(
        pl.kernel,
        mesh=mesh,
        compiler_params=pltpu.CompilerParams(
            needs_layout_passes=False, use_tc_tiling_on_sc=True),
        out_type=jax.ShapeDtypeStruct((PROWS * ROWW,), jnp.float32),
        scratch_types=[
            pltpu.VMEM((EMBED, WIN + 1), jnp.float32),
            pltpu.VMEM((EMBED, WIN + 1), jnp.float32),
            pltpu.VMEM((WINW,), jnp.float32),
            pltpu.VMEM((WINW,), jnp.float32),
            pltpu.SemaphoreType.DMA,
            pltpu.SemaphoreType.DMA,
            pltpu.SemaphoreType.DMA,
            pltpu.SemaphoreType.DMA,
        ],
    )(_relayout_body)
    wl = jnp.reshape(relayout(wt), (PROWS, ROWW))
    tailp = jnp.reshape(W[TAILBASE:, :], (PACK, ROWW))

    gather = functools.partial(
        pl.kernel,
        mesh=mesh,
        compiler_params=pltpu.CompilerParams(
            needs_layout_passes=False, use_tc_tiling_on_sc=False),
        out_type=(jax.ShapeDtypeStruct((BATCH,), jnp.float32),
                  jax.ShapeDtypeStruct((BATCH,), jnp.float32)),
        scratch_types=[
            pltpu.VMEM((CHUNK,), jnp.int32),
            pltpu.VMEM((CHUNK,), jnp.int32),
            pltpu.VMEM((CHUNK,), jnp.int32),
            pltpu.VMEM((CHUNK,), jnp.int32),
            pltpu.VMEM((CHUNK,), jnp.int32),
            pltpu.VMEM((CHUNK,), jnp.int32),
            pltpu.VMEM((CHUNK, ROWW), jnp.float32),
            pltpu.VMEM((CHUNK, ROWW), jnp.float32),
            pltpu.VMEM((CHUNK, ROWW), jnp.float32),
            pltpu.VMEM((PACK, ROWW), jnp.float32),
            pltpu.VMEM((BPW,), jnp.float32),
            pltpu.VMEM((BPW,), jnp.float32),
            pltpu.SemaphoreType.DMA,
        ],
    )(_gather_body)
    return gather(pos_u, pos_v, neg_v, wl, tailp)


# speedup vs baseline: 2.3036x; 2.2619x over previous
import functools

import jax
import jax.numpy as jnp
from jax import lax
from jax.experimental import pallas as pl
from jax.experimental.pallas import tpu as pltpu
from jax.experimental.pallas import tpu_sc as plsc

NODES = 1000000
BATCH = 16384
EMBED = 16
PACK = 8
ROWW = PACK * EMBED

_INFO = plsc.get_sparse_core_info()
NC = _INFO.num_cores
NS = _INFO.num_subcores
L = _INFO.num_lanes
NW = NC * NS
BPW = BATCH // NW
CHUNK = 128
NCHUNK = BPW // CHUNK
GROUPS = CHUNK // L

WIN = 512
NFULL = NODES // 128
NSUP = NODES // WIN
TAILBASE = NSUP * WIN
TPW = (NSUP + NW - 1) // NW
WINW = 16 * WIN
PROWS = NODES // PACK
JMAX = TAILBASE // PACK - 1


def _relayout_body(wt_hbm, wl_out, xb0, xb1, yb0, yb1, si0, si1, so0, so1):
    wid = lax.axis_index("s") * NC + lax.axis_index("c")
    dlane = lax.iota(jnp.int32, L)

    def win(c):
        return wt_hbm.at[:, pl.ds(pl.multiple_of(c * WIN, 128), WIN)]

    def outw(c):
        return wl_out.at[pl.ds(pl.multiple_of(c * WINW, WINW), WINW)]

    def start_in(c, xb, s):
        @pl.when(c < NSUP)
        def _():
            pltpu.async_copy(win(c), xb, s)

    def wait_in(c, xb, s):
        @pl.when(c < NSUP)
        def _():
            pltpu.make_async_copy(win(c), xb, s).wait()

    def permute_and_flush(c, xb, yb, s):
        @pl.when(c < NSUP)
        def _():
            @plsc.parallel_loop(0, WIN, unroll=16)
            def _(i):
                cols = (dlane + i) & (WIN - 1)
                vals = plsc.load_gather(xb, [dlane, cols])
                plsc.store_scatter(yb, [cols * EMBED + dlane], vals)

            pltpu.async_copy(yb, outw(c), s)

    def wait_out(cond, c, yb, s):
        @pl.when(cond)
        def _():
            pltpu.make_async_copy(yb, outw(jnp.minimum(c, NSUP - 1)), s).wait()

    start_in(wid, xb0, si0)

    def rnd(r, _):
        t0 = 2 * r
        c0 = wid + NW * t0
        c1 = c0 + NW
        c2 = c1 + NW
        start_in(c1, xb1, si1)
        wait_in(c0, xb0, si0)
        wait_out(r > 0, c0, yb0, so0)
        permute_and_flush(c0, xb0, yb0, so0)
        start_in(c2, xb0, si0)
        wait_in(c1, xb1, si1)
        wait_out(r > 0, c1, yb1, so1)
        permute_and_flush(c1, xb1, yb1, so1)
        return 0

    lax.fori_loop(0, (TPW + 1) // 2, rnd, 0)
    ceven = wid + NW * (TPW - 2)
    wait_out(ceven < NSUP, ceven, yb0, so0)
    codd = wid + NW * (TPW - 1)
    wait_out(codd < NSUP, codd, yb1, so1)


def _gather_body(pos_u_hbm, pos_v_hbm, neg_v_hbm, wl_hbm, tail_hbm,
                 pos_out, neg_out,
                 iu, iv, inn, ju, jv, jn, ubuf, vbuf, nbuf, tbuf,
                 acc_p, acc_n, sem):
    wid = lax.axis_index("s") * NC + lax.axis_index("c")
    base = wid * BPW
    lane = lax.iota(jnp.int32, L)
    pltpu.sync_copy(tail_hbm, tbuf)

    def chunk_body(k, _):
        cbase = base + k * CHUNK
        pltpu.sync_copy(pos_u_hbm.at[pl.ds(cbase, CHUNK)], iu)
        pltpu.sync_copy(pos_v_hbm.at[pl.ds(cbase, CHUNK)], iv)
        pltpu.sync_copy(neg_v_hbm.at[pl.ds(cbase, CHUNK)], inn)
        for g in range(GROUPS):
            sl = pl.ds(g * L, L)
            ju[sl] = jnp.minimum(lax.shift_right_logical(iu[sl], 3), JMAX)
            jv[sl] = jnp.minimum(lax.shift_right_logical(iv[sl], 3), JMAX)
            jn[sl] = jnp.minimum(lax.shift_right_logical(inn[sl], 3), JMAX)
        cu = pltpu.async_copy(wl_hbm.at[ju], ubuf, sem)
        cv = pltpu.async_copy(wl_hbm.at[jv], vbuf, sem)
        cn = pltpu.async_copy(wl_hbm.at[jn], nbuf, sem)
        cu.wait()
        cv.wait()
        cn.wait()

        def pick(buf, isl, ridx):
            b = (isl & 7) * EMBED
            tr = jnp.clip(lax.shift_right_logical(isl, 3) - (JMAX + 1),
                          0, PACK - 1)
            m = isl >= TAILBASE

            def at(d):
                return jnp.where(m,
                                 plsc.load_gather(tbuf, [tr, b + d]),
                                 plsc.load_gather(buf, [ridx, b + d]))
            return at

        for g in range(GROUPS):
            sl = pl.ds(g * L, L)
            ridx = g * L + lane
            gu = pick(ubuf, iu[sl], ridx)
            gv = pick(vbuf, iv[sl], ridx)
            gn = pick(nbuf, inn[sl], ridx)
            ap = jnp.zeros((L,), jnp.float32)
            an = jnp.zeros((L,), jnp.float32)
            for step in range(EMBED):
                dv = (lane + step) & (EMBED - 1)
                ud = gu(dv)
                ap = ap + ud * gv(dv)
                an = an + ud * gn(dv)
            acc_p[pl.ds(k * CHUNK + g * L, L)] = ap
            acc_n[pl.ds(k * CHUNK + g * L, L)] = an
        return 0

    lax.fori_loop(0, NCHUNK, chunk_body, 0)
    pltpu.sync_copy(acc_p, pos_out.at[pl.ds(base, BPW)])
    pltpu.sync_copy(acc_n, neg_out.at[pl.ds(base, BPW)])


@jax.jit
def kernel(pos_u, pos_v, neg_v, W):
    pos_u = pos_u.astype(jnp.int32)
    pos_v = pos_v.astype(jnp.int32)
    neg_v = neg_v.astype(jnp.int32)
    wt = W.T
    mesh = plsc.VectorSubcoreMesh(core_axis_name="c", subcore_axis_name="s")

    relayout = functools.partial(
        pl.kernel,
        mesh=mesh,
        compiler_params=pltpu.CompilerParams(
            needs_layout_passes=False, use_tc_tiling_on_sc=True),
        out_type=jax.ShapeDtypeStruct((PROWS * ROWW,), jnp.float32),
        scratch_types=[
            pltpu.VMEM((EMBED, WIN), jnp.float32),
            pltpu.VMEM((EMBED, WIN), jnp.float32),
            pltpu.VMEM((WINW,), jnp.float32),
            pltpu.VMEM((WINW,), jnp.float32),
            pltpu.SemaphoreType.DMA,
            pltpu.SemaphoreType.DMA,
            pltpu.SemaphoreType.DMA,
            pltpu.SemaphoreType.DMA,
        ],
    )(_relayout_body)
    wl = jnp.reshape(relayout(wt), (PROWS, ROWW))
    tailp = jnp.reshape(W[TAILBASE:, :], (PACK, ROWW))

    gather = functools.partial(
        pl.kernel,
        mesh=mesh,
        compiler_params=pltpu.CompilerParams(
            needs_layout_passes=False, use_tc_tiling_on_sc=False),
        out_type=(jax.ShapeDtypeStruct((BATCH,), jnp.float32),
                  jax.ShapeDtypeStruct((BATCH,), jnp.float32)),
        scratch_types=[
            pltpu.VMEM((CHUNK,), jnp.int32),
            pltpu.VMEM((CHUNK,), jnp.int32),
            pltpu.VMEM((CHUNK,), jnp.int32),
            pltpu.VMEM((CHUNK,), jnp.int32),
            pltpu.VMEM((CHUNK,), jnp.int32),
            pltpu.VMEM((CHUNK,), jnp.int32),
            pltpu.VMEM((CHUNK, ROWW), jnp.float32),
            pltpu.VMEM((CHUNK, ROWW), jnp.float32),
            pltpu.VMEM((CHUNK, ROWW), jnp.float32),
            pltpu.VMEM((PACK, ROWW), jnp.float32),
            pltpu.VMEM((BPW,), jnp.float32),
            pltpu.VMEM((BPW,), jnp.float32),
            pltpu.SemaphoreType.DMA,
        ],
    )(_gather_body)
    return gather(pos_u, pos_v, neg_v, wl, tailp)


# speedup vs baseline: 2.4669x; 1.0709x over previous
import functools

import jax
import jax.numpy as jnp
from jax import lax
from jax.experimental import pallas as pl
from jax.experimental.pallas import tpu as pltpu
from jax.experimental.pallas import tpu_sc as plsc

NODES = 1000000
BATCH = 16384
EMBED = 16
PACK = 8
ROWW = PACK * EMBED

_INFO = plsc.get_sparse_core_info()
NC = _INFO.num_cores
NS = _INFO.num_subcores
L = _INFO.num_lanes
NW = NC * NS
BPW = BATCH // NW
CHUNK = 128
NCHUNK = BPW // CHUNK
GROUPS = CHUNK // L

WIN = 512
NFULL = NODES // 128
NSUP = NODES // WIN
TAILBASE = NSUP * WIN
TPW = (NSUP + NW - 1) // NW
WINW = 16 * WIN
PROWS = NODES // PACK
JMAX = TAILBASE // PACK - 1


def _relayout_body(wt_hbm, wl_out, xb0, xb1, xb2, xb3, yb0, yb1, yb2, yb3,
                   si0, si1, si2, si3, so0, so1, so2, so3):
    wid = lax.axis_index("s") * NC + lax.axis_index("c")
    dlane = lax.iota(jnp.int32, L)

    def win(c):
        return wt_hbm.at[:, pl.ds(pl.multiple_of(c * WIN, 128), WIN)]

    def outw(c):
        return wl_out.at[pl.ds(pl.multiple_of(c * WINW, WINW), WINW)]

    def start_in(c, xb, s):
        @pl.when(c < NSUP)
        def _():
            pltpu.async_copy(win(c), xb, s)

    def wait_in(c, xb, s):
        @pl.when(c < NSUP)
        def _():
            pltpu.make_async_copy(win(c), xb, s).wait()

    def permute_and_flush(c, xb, yb, s):
        @pl.when(c < NSUP)
        def _():
            @plsc.parallel_loop(0, WIN, unroll=16)
            def _(i):
                cols = (dlane + i) & (WIN - 1)
                vals = plsc.load_gather(xb, [dlane, cols])
                plsc.store_scatter(yb, [cols * EMBED + dlane], vals)

            pltpu.async_copy(yb, outw(c), s)

    def wait_out(cond, c, yb, s):
        @pl.when(cond)
        def _():
            pltpu.make_async_copy(yb, outw(jnp.minimum(c, NSUP - 1)), s).wait()

    XB = [xb0, xb1, xb2, xb3]
    YB = [yb0, yb1, yb2, yb3]
    SI = [si0, si1, si2, si3]
    SO = [so0, so1, so2, so3]
    DEPTH = 4
    ROUNDS = (TPW + DEPTH - 1) // DEPTH

    for j in range(DEPTH):
        start_in(wid + NW * j, XB[j], SI[j])

    def rnd(r, _):
        for j in range(DEPTH):
            c = wid + NW * (DEPTH * r + j)
            wait_in(c, XB[j], SI[j])
            wait_out(r > 0, c, YB[j], SO[j])
            permute_and_flush(c, XB[j], YB[j], SO[j])
            start_in(c + NW * DEPTH, XB[j], SI[j])
        return 0

    lax.fori_loop(0, ROUNDS, rnd, 0)
    for j in range(DEPTH):
        c = wid + NW * (DEPTH * (ROUNDS - 1) + j)
        wait_out(c < NSUP, c, YB[j], SO[j])


def _gather_body(pos_u_hbm, pos_v_hbm, neg_v_hbm, wl_hbm, tail_hbm,
                 pos_out, neg_out,
                 iu, iv, inn, ju, jv, jn, ubuf, vbuf, nbuf, tbuf,
                 acc_p, acc_n, sem):
    wid = lax.axis_index("s") * NC + lax.axis_index("c")
    base = wid * BPW
    lane = lax.iota(jnp.int32, L)
    pltpu.sync_copy(tail_hbm, tbuf)

    def chunk_body(k, _):
        cbase = base + k * CHUNK
        pltpu.sync_copy(pos_u_hbm.at[pl.ds(cbase, CHUNK)], iu)
        pltpu.sync_copy(pos_v_hbm.at[pl.ds(cbase, CHUNK)], iv)
        pltpu.sync_copy(neg_v_hbm.at[pl.ds(cbase, CHUNK)], inn)
        for g in range(GROUPS):
            sl = pl.ds(g * L, L)
            ju[sl] = jnp.minimum(lax.shift_right_logical(iu[sl], 3), JMAX)
            jv[sl] = jnp.minimum(lax.shift_right_logical(iv[sl], 3), JMAX)
            jn[sl] = jnp.minimum(lax.shift_right_logical(inn[sl], 3), JMAX)
        cu = pltpu.async_copy(wl_hbm.at[ju], ubuf, sem)
        cv = pltpu.async_copy(wl_hbm.at[jv], vbuf, sem)
        cn = pltpu.async_copy(wl_hbm.at[jn], nbuf, sem)
        cu.wait()
        cv.wait()
        cn.wait()

        def pick(buf, isl, ridx):
            b = (isl & 7) * EMBED
            tr = jnp.clip(lax.shift_right_logical(isl, 3) - (JMAX + 1),
                          0, PACK - 1)
            m = isl >= TAILBASE

            def at(d):
                return jnp.where(m,
                                 plsc.load_gather(tbuf, [tr, b + d]),
                                 plsc.load_gather(buf, [ridx, b + d]))
            return at

        for g in range(GROUPS):
            sl = pl.ds(g * L, L)
            ridx = g * L + lane
            gu = pick(ubuf, iu[sl], ridx)
            gv = pick(vbuf, iv[sl], ridx)
            gn = pick(nbuf, inn[sl], ridx)
            ap = jnp.zeros((L,), jnp.float32)
            an = jnp.zeros((L,), jnp.float32)
            for step in range(EMBED):
                dv = (lane + step) & (EMBED - 1)
                ud = gu(dv)
                ap = ap + ud * gv(dv)
                an = an + ud * gn(dv)
            acc_p[pl.ds(k * CHUNK + g * L, L)] = ap
            acc_n[pl.ds(k * CHUNK + g * L, L)] = an
        return 0

    lax.fori_loop(0, NCHUNK, chunk_body, 0)
    pltpu.sync_copy(acc_p, pos_out.at[pl.ds(base, BPW)])
    pltpu.sync_copy(acc_n, neg_out.at[pl.ds(base, BPW)])


@jax.jit
def kernel(pos_u, pos_v, neg_v, W):
    pos_u = pos_u.astype(jnp.int32)
    pos_v = pos_v.astype(jnp.int32)
    neg_v = neg_v.astype(jnp.int32)
    wt = W.T
    mesh = plsc.VectorSubcoreMesh(core_axis_name="c", subcore_axis_name="s")

    relayout = functools.partial(
        pl.kernel,
        mesh=mesh,
        compiler_params=pltpu.CompilerParams(
            needs_layout_passes=False, use_tc_tiling_on_sc=True),
        out_type=jax.ShapeDtypeStruct((PROWS * ROWW,), jnp.float32),
        scratch_types=[
            pltpu.VMEM((EMBED, WIN), jnp.float32),
            pltpu.VMEM((EMBED, WIN), jnp.float32),
            pltpu.VMEM((EMBED, WIN), jnp.float32),
            pltpu.VMEM((EMBED, WIN), jnp.float32),
            pltpu.VMEM((WINW,), jnp.float32),
            pltpu.VMEM((WINW,), jnp.float32),
            pltpu.VMEM((WINW,), jnp.float32),
            pltpu.VMEM((WINW,), jnp.float32),
            pltpu.SemaphoreType.DMA,
            pltpu.SemaphoreType.DMA,
            pltpu.SemaphoreType.DMA,
            pltpu.SemaphoreType.DMA,
            pltpu.SemaphoreType.DMA,
            pltpu.SemaphoreType.DMA,
            pltpu.SemaphoreType.DMA,
            pltpu.SemaphoreType.DMA,
        ],
    )(_relayout_body)
    wl = jnp.reshape(relayout(wt), (PROWS, ROWW))
    tailp = jnp.reshape(W[TAILBASE:, :], (PACK, ROWW))

    gather = functools.partial(
        pl.kernel,
        mesh=mesh,
        compiler_params=pltpu.CompilerParams(
            needs_layout_passes=False, use_tc_tiling_on_sc=False),
        out_type=(jax.ShapeDtypeStruct((BATCH,), jnp.float32),
                  jax.ShapeDtypeStruct((BATCH,), jnp.float32)),
        scratch_types=[
            pltpu.VMEM((CHUNK,), jnp.int32),
            pltpu.VMEM((CHUNK,), jnp.int32),
            pltpu.VMEM((CHUNK,), jnp.int32),
            pltpu.VMEM((CHUNK,), jnp.int32),
            pltpu.VMEM((CHUNK,), jnp.int32),
            pltpu.VMEM((CHUNK,), jnp.int32),
            pltpu.VMEM((CHUNK, ROWW), jnp.float32),
            pltpu.VMEM((CHUNK, ROWW), jnp.float32),
            pltpu.VMEM((CHUNK, ROWW), jnp.float32),
            pltpu.VMEM((PACK, ROWW), jnp.float32),
            pltpu.VMEM((BPW,), jnp.float32),
            pltpu.VMEM((BPW,), jnp.float32),
            pltpu.SemaphoreType.DMA,
        ],
    )(_gather_body)
    return gather(pos_u, pos_v, neg_v, wl, tailp)
